# baseline (speedup 1.0000x reference)
import functools

import jax
import jax.numpy as jnp
import numpy as np
from jax import lax
from jax.experimental import pallas as pl
from jax.experimental.pallas import tpu as pltpu
from jax.experimental.pallas import tpu_sc as plsc

N = 10000
E = 320000
D = 128

NC = 2
NS = 16
TILES = NC * NS
CH = 64
NCHUNK = 160
SLAB = 40
EPAD = TILES * NCHUNK * CH
RP0 = 624
ZROWS = 16
LANES = 16
DV = D // LANES
DW = D // 32

ROW_BLK = 1000

_COLS_A = np.concatenate([np.arange(32 * j, 32 * j + 16) for j in range(DW)])
_COLS_B = _COLS_A + 16


def _bf16_bits(x):
    u = lax.bitcast_convert_type(x, jnp.int32)
    return (u + 0x7FFF + ((u >> 16) & 1)) >> 16


def _dense_body(x_ref, wt_ref, bt_ref, wa_ref, ba_ref, wb_ref, bb_ref,
                t_ref, sb_ref):
    x = x_ref[...]
    x2 = x * x
    xc = jnp.concatenate([x, x2], axis=1)
    t = jnp.dot(x, wt_ref[...], preferred_element_type=jnp.float32) + bt_ref[...]
    sa = jnp.dot(xc, wa_ref[...], preferred_element_type=jnp.float32) + ba_ref[...]
    sb = jnp.dot(xc, wb_ref[...], preferred_element_type=jnp.float32) + bb_ref[...]
    t_ref[...] = t
    sb_ref[...] = ((_bf16_bits(sb) << 16) | (_bf16_bits(sa) & 0xFFFF))


def _dense_call(x, wtT, bt2, wA, bA2, wB, bB2):
    wspec = pl.BlockSpec((D, D), lambda i: (0, 0))
    w2spec = pl.BlockSpec((2 * D, D // 2), lambda i: (0, 0))
    bspec = pl.BlockSpec((1, D), lambda i: (0, 0))
    b2spec = pl.BlockSpec((1, D // 2), lambda i: (0, 0))
    nspec = pl.BlockSpec((ROW_BLK, D), lambda i: (i, 0))
    hspec = pl.BlockSpec((ROW_BLK, D // 2), lambda i: (i, 0))
    return pl.pallas_call(
        _dense_body,
        grid=(N // ROW_BLK,),
        in_specs=[nspec, wspec, bspec, w2spec, b2spec, w2spec, b2spec],
        out_specs=[nspec, hspec],
        out_shape=[jax.ShapeDtypeStruct((N, D), jnp.float32),
                   jax.ShapeDtypeStruct((N, D // 2), jnp.int32)],
    )(x, wtT, bt2, wA, bA2, wB, bB2)


def _add_body(a_ref, b_ref, o_ref):
    o_ref[...] = a_ref[...] + b_ref[...]


def _add_call(a, b):
    nspec = pl.BlockSpec((ROW_BLK, D), lambda i: (i, 0))
    return pl.pallas_call(
        _add_body,
        grid=(N // ROW_BLK,),
        in_specs=[nspec, nspec],
        out_specs=nspec,
        out_shape=jax.ShapeDtypeStruct((N, D), jnp.float32),
    )(a, b)


def _sc_body(s_hbm, t_hbm, src_hbm, dst_hbm, w_hbm, p0_hbm, p1_hbm,
             src_v, dst_v, w_v, rows_b, rows_f, acc, sem, sem_s):
    c = lax.axis_index("c")
    sid = lax.axis_index("s")
    wid = c * NS + sid
    r0 = pl.multiple_of(sid * RP0, 8)

    @pl.when(c == 0)
    def _():
        pltpu.sync_copy(t_hbm.at[pl.ds(r0, RP0)], acc.at[pl.ds(r0, RP0)])

        @pl.when(sid == NS - 1)
        def _():
            pltpu.sync_copy(t_hbm.at[pl.ds(NS * RP0, N - NS * RP0)],
                            acc.at[pl.ds(NS * RP0, N - NS * RP0)])

    @pl.when(c == 1)
    def _():
        zv = jnp.zeros((LANES,), jnp.float32)
        for r in range(ZROWS):
            for j in range(DV):
                rows_f[0, r, pl.ds(j * LANES, LANES)] = zv

        def zcopy(q, carry):
            off = pl.multiple_of(r0 + q * ZROWS, 8)
            pltpu.make_async_copy(rows_f.at[0, pl.ds(0, ZROWS)],
                                  acc.at[pl.ds(off, ZROWS)], sem_s).start()
            return carry

        lax.fori_loop(0, RP0 // ZROWS, zcopy, 0)

        @pl.when(sid == NS - 1)
        def _():
            pltpu.sync_copy(rows_f.at[0, pl.ds(0, ZROWS)],
                            acc.at[pl.ds(NS * RP0, N - NS * RP0)])

        def zdrain(q, carry):
            off = pl.multiple_of(r0 + q * ZROWS, 8)
            pltpu.make_async_copy(rows_f.at[0, pl.ds(0, ZROWS)],
                                  acc.at[pl.ds(off, ZROWS)], sem_s).wait()
            return carry

        lax.fori_loop(0, RP0 // ZROWS, zdrain, 0)

    plsc.subcore_barrier()

    def _mult(b, k):
        himask = jnp.full((LANES,), -65536, dtype=jnp.int32)

        def gbody(g, carry):
            wvec = w_v[k, pl.ds(g * LANES, LANES)]
            e0 = g * LANES
            for ei in range(LANES):
                wt = wvec[ei]
                for j in range(DW):
                    u = rows_b[b, e0 + ei, pl.ds(j * LANES, LANES)]
                    lo = lax.bitcast_convert_type(
                        lax.shift_left(u, 16), jnp.float32)
                    hi = lax.bitcast_convert_type(u & himask, jnp.float32)
                    rows_f[b, e0 + ei, pl.ds(j * 32, LANES)] = lo * wt
                    rows_f[b, e0 + ei, pl.ds(j * 32 + LANES, LANES)] = hi * wt
            return carry
        lax.fori_loop(0, CH // LANES, gbody, 0)

    for h in range(NCHUNK // SLAB):
        pltpu.sync_copy(src_hbm.at[wid, pl.ds(h * SLAB, SLAB)], src_v)
        pltpu.sync_copy(dst_hbm.at[wid, pl.ds(h * SLAB, SLAB)], dst_v)
        pltpu.sync_copy(w_hbm.at[wid, pl.ds(h * SLAB, SLAB)], w_v)

        pltpu.async_copy(s_hbm.at[src_v.at[0]], rows_b.at[0], sem)

        def pair(k2, carry):
            for b in range(2):
                k = 2 * k2 + b
                pltpu.make_async_copy(
                    s_hbm.at[src_v.at[k]], rows_b.at[b], sem).wait()
                if b == 0:
                    pltpu.async_copy(
                        s_hbm.at[src_v.at[k + 1]], rows_b.at[1], sem)
                else:
                    @pl.when(k2 < SLAB // 2 - 1)
                    def _():
                        pltpu.async_copy(
                            s_hbm.at[src_v.at[k + 1]], rows_b.at[0], sem)

                @pl.when(k2 > 0)
                def _():
                    pltpu.make_async_copy(
                        rows_f.at[b], acc.at[dst_v.at[k - 2]], sem_s).wait()

                _mult(b, k)
                pltpu.make_async_copy(
                    rows_f.at[b], acc.at[dst_v.at[k]], sem_s).start(add=True)
            return carry

        lax.fori_loop(0, SLAB // 2, pair, 0)
        pltpu.make_async_copy(
            rows_f.at[0], acc.at[dst_v.at[SLAB - 2]], sem_s).wait()
        pltpu.make_async_copy(
            rows_f.at[1], acc.at[dst_v.at[SLAB - 1]], sem_s).wait()

    plsc.subcore_barrier()

    @pl.when(c == 0)
    def _():
        pltpu.sync_copy(acc.at[pl.ds(r0, RP0)], p0_hbm.at[pl.ds(r0, RP0)])

        @pl.when(sid == NS - 1)
        def _():
            pltpu.sync_copy(acc.at[pl.ds(NS * RP0, N - NS * RP0)],
                            p0_hbm.at[pl.ds(NS * RP0, N - NS * RP0)])

    @pl.when(c == 1)
    def _():
        pltpu.sync_copy(acc.at[pl.ds(r0, RP0)], p1_hbm.at[pl.ds(r0, RP0)])

        @pl.when(sid == NS - 1)
        def _():
            pltpu.sync_copy(acc.at[pl.ds(NS * RP0, N - NS * RP0)],
                            p1_hbm.at[pl.ds(NS * RP0, N - NS * RP0)])


def _sc_call(s_b, t, src3, dst3, w3):
    mesh = plsc.VectorSubcoreMesh(
        core_axis_name="c", subcore_axis_name="s",
        num_cores=NC, num_subcores=NS)
    f = pl.kernel(
        _sc_body,
        out_type=[jax.ShapeDtypeStruct((N, D), jnp.float32)] * 2,
        mesh=mesh,
        scratch_types=[
            pltpu.VMEM((SLAB, CH), jnp.int32),
            pltpu.VMEM((SLAB, CH), jnp.int32),
            pltpu.VMEM((SLAB, CH), jnp.float32),
            pltpu.VMEM((2, CH, D // 2), jnp.int32),
            pltpu.VMEM((2, CH, D), jnp.float32),
            pltpu.VMEM_SHARED((N, D), jnp.float32),
            pltpu.SemaphoreType.DMA,
            pltpu.SemaphoreType.DMA,
        ],
        compiler_params=pltpu.CompilerParams(use_tc_tiling_on_sc=False),
    )
    return f(s_b, t, src3, dst3, w3)


def kernel(features, edge_index, edge_weight, W_t, b_t, W_i, b_i):
    colsA = jnp.asarray(_COLS_A)
    colsB = jnp.asarray(_COLS_B)
    wtT = W_t.T
    wiT = W_i.T
    bs = b_t + b_i
    wA = jnp.concatenate([wtT[:, colsA], wiT[:, colsA]], axis=0)
    wB = jnp.concatenate([wtT[:, colsB], wiT[:, colsB]], axis=0)
    t, s_b = _dense_call(features, wtT, b_t[None, :],
                         wA, bs[colsA][None, :], wB, bs[colsB][None, :])
    pad = EPAD - E
    zi = jnp.arange(pad, dtype=jnp.int32) % N
    src3 = jnp.concatenate([edge_index[0], zi]).reshape(TILES, NCHUNK, CH)
    dst3 = jnp.concatenate([edge_index[1], zi]).reshape(TILES, NCHUNK, CH)
    w3 = jnp.concatenate([edge_weight, jnp.zeros((pad,), jnp.float32)]
                         ).reshape(TILES, NCHUNK, CH)
    p0, p1 = _sc_call(s_b, t, src3, dst3, w3)
    return _add_call(p0, p1)


# speedup vs baseline: 1.7673x; 1.7673x over previous
import functools

import jax
import jax.numpy as jnp
from jax import lax
from jax.experimental import pallas as pl
from jax.experimental.pallas import tpu as pltpu
from jax.experimental.pallas import tpu_sc as plsc

N = 10000
E = 320000
D = 128

NC = 2
NS = 16
TILES = NC * NS
CH = 128
NCHUNK = 80
HCH = 40
EPAD = TILES * NCHUNK * CH
RP0 = 624
ZROWS = 16
LANES = 16
DV = D // LANES

ROW_BLK = 1000


def _mmT(x, w):
    return lax.dot_general(x, w, (((1,), (1,)), ((), ())),
                           preferred_element_type=jnp.float32)


def _dense_body(x_ref, wt_ref, bt_ref, wi_ref, bi_ref, t_ref, s_ref):
    x = x_ref[...]
    t = _mmT(x, wt_ref[...]) + bt_ref[...]
    i = _mmT(x * x, wi_ref[...]) + bi_ref[...]
    t_ref[...] = t
    s_ref[...] = t + i


def _dense_call(x, wtT, bt2, wiT, bi2):
    return pl.pallas_call(
        _dense_body,
        grid=(N // ROW_BLK,),
        in_specs=[
            pl.BlockSpec((ROW_BLK, D), lambda i: (i, 0)),
            pl.BlockSpec((D, D), lambda i: (0, 0)),
            pl.BlockSpec((1, D), lambda i: (0, 0)),
            pl.BlockSpec((D, D), lambda i: (0, 0)),
            pl.BlockSpec((1, D), lambda i: (0, 0)),
        ],
        out_specs=[
            pl.BlockSpec((ROW_BLK, D), lambda i: (i, 0)),
            pl.BlockSpec((ROW_BLK, D), lambda i: (i, 0)),
        ],
        out_shape=[jax.ShapeDtypeStruct((N, D), jnp.float32)] * 2,
    )(x, wtT, bt2, wiT, bi2)


def _add_body(a_ref, b_ref, o_ref):
    o_ref[...] = a_ref[...] + b_ref[...]


def _add_call(a, b):
    return pl.pallas_call(
        _add_body,
        grid=(N // ROW_BLK,),
        in_specs=[
            pl.BlockSpec((ROW_BLK, D), lambda i: (i, 0)),
            pl.BlockSpec((ROW_BLK, D), lambda i: (i, 0)),
        ],
        out_specs=pl.BlockSpec((ROW_BLK, D), lambda i: (i, 0)),
        out_shape=jax.ShapeDtypeStruct((N, D), jnp.float32),
    )(a, b)


def _sc_body(s_hbm, t_hbm, src_hbm, dst_hbm, w_hbm, p0_hbm, p1_hbm,
             src_v, dst_v, w_v, rows_v, acc, sem, sem_s):
    c = lax.axis_index("c")
    sid = lax.axis_index("s")
    wid = c * NS + sid
    r0 = pl.multiple_of(sid * RP0, 8)

    @pl.when(c == 0)
    def _():
        pltpu.sync_copy(t_hbm.at[pl.ds(r0, RP0)], acc.at[pl.ds(r0, RP0)])

        @pl.when(sid == NS - 1)
        def _():
            pltpu.sync_copy(t_hbm.at[pl.ds(NS * RP0, N - NS * RP0)],
                            acc.at[pl.ds(NS * RP0, N - NS * RP0)])

    @pl.when(c == 1)
    def _():
        zv = jnp.zeros((LANES,), jnp.float32)
        for r in range(ZROWS):
            for j in range(DV):
                rows_v[0, r, pl.ds(j * LANES, LANES)] = zv

        def zcopy(q, carry):
            off = pl.multiple_of(r0 + q * ZROWS, 8)
            pltpu.make_async_copy(rows_v.at[0, pl.ds(0, ZROWS)],
                                  acc.at[pl.ds(off, ZROWS)], sem_s).start()
            return carry

        lax.fori_loop(0, RP0 // ZROWS, zcopy, 0)

        @pl.when(sid == NS - 1)
        def _():
            pltpu.sync_copy(rows_v.at[0, pl.ds(0, ZROWS)],
                            acc.at[pl.ds(NS * RP0, N - NS * RP0)])

        def zdrain(q, carry):
            off = pl.multiple_of(r0 + q * ZROWS, 8)
            pltpu.make_async_copy(rows_v.at[0, pl.ds(0, ZROWS)],
                                  acc.at[pl.ds(off, ZROWS)], sem_s).wait()
            return carry

        lax.fori_loop(0, RP0 // ZROWS, zdrain, 0)

    plsc.subcore_barrier()

    def _mult(b, k):
        def gbody(g, carry):
            wvec = w_v[k, pl.ds(g * LANES, LANES)]
            e0 = g * LANES
            for ei in range(LANES):
                wt = wvec[ei]
                for j in range(DV):
                    sl = pl.ds(j * LANES, LANES)
                    rows_v[b, e0 + ei, sl] = rows_v[b, e0 + ei, sl] * wt
            return carry
        lax.fori_loop(0, CH // LANES, gbody, 0)

    for h in range(NCHUNK // HCH):
        pltpu.sync_copy(src_hbm.at[wid, pl.ds(h * HCH, HCH)], src_v)
        pltpu.sync_copy(dst_hbm.at[wid, pl.ds(h * HCH, HCH)], dst_v)
        pltpu.sync_copy(w_hbm.at[wid, pl.ds(h * HCH, HCH)], w_v)

        pltpu.async_copy(s_hbm.at[src_v.at[0]], rows_v.at[0], sem)

        def pair(k2, carry):
            for b in range(2):
                k = 2 * k2 + b
                pltpu.make_async_copy(
                    s_hbm.at[src_v.at[k]], rows_v.at[b], sem).wait()
                if b == 0:
                    @pl.when(k2 > 0)
                    def _():
                        pltpu.make_async_copy(
                            rows_v.at[1], acc.at[dst_v.at[k - 1]],
                            sem_s).wait()
                    pltpu.async_copy(
                        s_hbm.at[src_v.at[k + 1]], rows_v.at[1], sem)
                else:
                    pltpu.make_async_copy(
                        rows_v.at[0], acc.at[dst_v.at[k - 1]], sem_s).wait()

                    @pl.when(k2 < HCH // 2 - 1)
                    def _():
                        pltpu.async_copy(
                            s_hbm.at[src_v.at[k + 1]], rows_v.at[0], sem)

                _mult(b, k)
                pltpu.make_async_copy(
                    rows_v.at[b], acc.at[dst_v.at[k]], sem_s).start(add=True)
            return carry

        lax.fori_loop(0, HCH // 2, pair, 0)
        pltpu.make_async_copy(
            rows_v.at[1], acc.at[dst_v.at[HCH - 1]], sem_s).wait()

    plsc.subcore_barrier()

    @pl.when(c == 0)
    def _():
        pltpu.sync_copy(acc.at[pl.ds(r0, RP0)], p0_hbm.at[pl.ds(r0, RP0)])

        @pl.when(sid == NS - 1)
        def _():
            pltpu.sync_copy(acc.at[pl.ds(NS * RP0, N - NS * RP0)],
                            p0_hbm.at[pl.ds(NS * RP0, N - NS * RP0)])

    @pl.when(c == 1)
    def _():
        pltpu.sync_copy(acc.at[pl.ds(r0, RP0)], p1_hbm.at[pl.ds(r0, RP0)])

        @pl.when(sid == NS - 1)
        def _():
            pltpu.sync_copy(acc.at[pl.ds(NS * RP0, N - NS * RP0)],
                            p1_hbm.at[pl.ds(NS * RP0, N - NS * RP0)])


def _sc_call(s, t, src3, dst3, w3, interpret=False):
    mesh = plsc.VectorSubcoreMesh(
        core_axis_name="c", subcore_axis_name="s",
        num_cores=NC, num_subcores=NS)
    f = pl.kernel(
        _sc_body,
        out_type=[jax.ShapeDtypeStruct((N, D), jnp.float32)] * 2,
        mesh=mesh,
        scratch_types=[
            pltpu.VMEM((HCH, CH), jnp.int32),
            pltpu.VMEM((HCH, CH), jnp.int32),
            pltpu.VMEM((HCH, CH), jnp.float32),
            pltpu.VMEM((2, CH, D), jnp.float32),
            pltpu.VMEM_SHARED((N, D), jnp.float32),
            pltpu.SemaphoreType.DMA,
            pltpu.SemaphoreType.DMA,
        ],
        interpret=interpret,
    )
    return f(s, t, src3, dst3, w3)


def kernel(features, edge_index, edge_weight, W_t, b_t, W_i, b_i):
    t, s = _dense_call(features, W_t, b_t[None, :], W_i, b_i[None, :])
    pad = EPAD - E
    ei_p = jnp.concatenate([edge_index, edge_index[:, :pad]], axis=1)
    src3 = ei_p[0].reshape(TILES, NCHUNK, CH)
    dst3 = ei_p[1].reshape(TILES, NCHUNK, CH)
    w3 = jnp.concatenate([edge_weight, jnp.zeros((pad,), jnp.float32)]
                         ).reshape(TILES, NCHUNK, CH)
    p0, p1 = _sc_call(s, t, src3, dst3, w3)
    return _add_call(p0, p1)
